# Initial kernel scaffold; baseline (speedup 1.0000x reference)
import jax
import jax.numpy as jnp
from jax.experimental import pallas as pl


def kernel(x, edge_index, W1, b1, W2, b2):
    raise NotImplementedError("write your pallas kernel here")


# speedup vs baseline: 15.8879x; 15.8879x over previous
import functools

import jax
import jax.numpy as jnp
from jax import lax
from jax.experimental import pallas as pl
from jax.experimental.pallas import tpu as pltpu
from jax.experimental.pallas import tpu_sc as plsc

N = 10000
DIN = 128
DH = 32
NCLS = 2
E = 320000

NC = 2
NS = 16
NW = NC * NS
EPT = E // NW
CHUNK = 80
NCHUNKS = EPT // CHUNK

NDEG = 10240
DEG_SL = NDEG // NS
NACC = 10240
ROW_SL = NACC // NS

_sc_mesh = plsc.VectorSubcoreMesh(
    core_axis_name="c", subcore_axis_name="s", num_cores=NC, num_subcores=NS
)
_sc_params = pltpu.CompilerParams(use_tc_tiling_on_sc=False)


@functools.partial(
    pl.kernel,
    out_type=jax.ShapeDtypeStruct((NC, NDEG), jnp.float32),
    mesh=_sc_mesh,
    compiler_params=_sc_params,
    scratch_types=[
        pltpu.VMEM_SHARED((NDEG,), jnp.float32),
        pltpu.VMEM((CHUNK,), jnp.int32),
        pltpu.VMEM((CHUNK,), jnp.float32),
    ],
)
def _deg_kernel(dst_hbm, zeros1_hbm, ones_hbm, deg_out, deg_sp, idx_v, ones_v):
    c = lax.axis_index("c")
    s = lax.axis_index("s")
    base = (c * NS + s) * EPT
    pltpu.sync_copy(
        zeros1_hbm.at[pl.ds(s * DEG_SL, DEG_SL)],
        deg_sp.at[pl.ds(s * DEG_SL, DEG_SL)],
    )
    pltpu.sync_copy(ones_hbm, ones_v)
    plsc.subcore_barrier()

    def body(ch, carry):
        pltpu.sync_copy(dst_hbm.at[pl.ds(base + ch * CHUNK, CHUNK)], idx_v)
        pltpu.sync_copy(ones_v, deg_sp.at[idx_v], add=True)
        return carry

    lax.fori_loop(0, NCHUNKS, body, 0)
    plsc.subcore_barrier()
    pltpu.sync_copy(
        deg_sp.at[pl.ds(s * DEG_SL, DEG_SL)],
        deg_out.at[c].at[pl.ds(s * DEG_SL, DEG_SL)],
    )


@functools.partial(
    pl.kernel,
    out_type=jax.ShapeDtypeStruct((NC, NACC, DH), jnp.float32),
    mesh=_sc_mesh,
    compiler_params=_sc_params,
    scratch_types=[
        pltpu.VMEM_SHARED((NACC, DH), jnp.float32),
        pltpu.VMEM((CHUNK,), jnp.int32),
        pltpu.VMEM((CHUNK,), jnp.int32),
        pltpu.VMEM((CHUNK, DH), jnp.float32),
        pltpu.SemaphoreType.DMA,
    ],
)
def _msg_kernel(hs_hbm, src_hbm, dst_hbm, zeros2_hbm, acc_out,
                acc_sp, src_v, dst_v, rows_v, sem):
    c = lax.axis_index("c")
    s = lax.axis_index("s")
    base = (c * NS + s) * EPT
    pltpu.sync_copy(
        zeros2_hbm.at[pl.ds(s * ROW_SL, ROW_SL)],
        acc_sp.at[pl.ds(s * ROW_SL, ROW_SL)],
    )
    plsc.subcore_barrier()

    def body(ch, carry):
        pltpu.sync_copy(src_hbm.at[pl.ds(base + ch * CHUNK, CHUNK)], src_v)
        pltpu.sync_copy(dst_hbm.at[pl.ds(base + ch * CHUNK, CHUNK)], dst_v)
        pltpu.async_copy(hs_hbm.at[src_v], rows_v, sem).wait()
        pltpu.sync_copy(rows_v, acc_sp.at[dst_v], add=True)
        return carry

    lax.fori_loop(0, NCHUNKS, body, 0)
    plsc.subcore_barrier()
    pltpu.sync_copy(
        acc_sp.at[pl.ds(s * ROW_SL, ROW_SL)],
        acc_out.at[c].at[pl.ds(s * ROW_SL, ROW_SL)],
    )


def _layer1_body(x_ref, w1_ref, degcol_ref, hs1_ref, dcol_ref):
    d = lax.rsqrt(degcol_ref[...])
    h = jnp.dot(x_ref[...], w1_ref[...], preferred_element_type=jnp.float32)
    hs1_ref[...] = h * d
    dcol_ref[...] = d


def _mid_body(acc_ref, hs1_ref, dcol_ref, b1_ref, hs2_ref):
    m = (acc_ref[0, :N] + acc_ref[1, :N] + hs1_ref[...]) * dcol_ref[...]
    out1 = jnp.maximum(m + b1_ref[...], 0.0)
    hs2_ref[...] = out1 * dcol_ref[...]


def _final_body(acc_ref, hs2_ref, dcol_ref, w2_ref, b2_ref, out_ref):
    p2 = (acc_ref[0, :N] + acc_ref[1, :N] + hs2_ref[...]) * dcol_ref[...]
    logits = (
        jnp.dot(p2, w2_ref[...], preferred_element_type=jnp.float32)
        + b2_ref[...]
    )
    l0 = logits[:, 0:1]
    l1 = logits[:, 1:2]
    mx = jnp.maximum(l0, l1)
    lse = mx + jnp.log(jnp.exp(l0 - mx) + jnp.exp(l1 - mx))
    out_ref[...] = logits - lse


def kernel(x, edge_index, W1, b1, W2, b2):
    src = edge_index[0]
    dst = edge_index[1]
    zeros1 = jnp.zeros((NDEG,), jnp.float32)
    zeros2 = jnp.zeros((NACC, DH), jnp.float32)
    ones = jnp.ones((CHUNK,), jnp.float32)

    degp = _deg_kernel(dst, zeros1, ones)
    degcol = (degp[0, :N] + degp[1, :N] + 1.0)[:, None]

    hs1, dcol = pl.pallas_call(
        _layer1_body,
        out_shape=[
            jax.ShapeDtypeStruct((N, DH), jnp.float32),
            jax.ShapeDtypeStruct((N, 1), jnp.float32),
        ],
    )(x, W1, degcol)

    acc1 = _msg_kernel(hs1, src, dst, zeros2)

    hs2 = pl.pallas_call(
        _mid_body,
        out_shape=jax.ShapeDtypeStruct((N, DH), jnp.float32),
    )(acc1, hs1, dcol, b1[None, :])

    acc2 = _msg_kernel(hs2, src, dst, zeros2)

    out = pl.pallas_call(
        _final_body,
        out_shape=jax.ShapeDtypeStruct((N, NCLS), jnp.float32),
    )(acc2, hs2, dcol, W2, b2[None, :])
    return out


# speedup vs baseline: 55.8679x; 3.5164x over previous
import functools

import jax
import jax.numpy as jnp
from jax import lax
from jax.experimental import pallas as pl
from jax.experimental.pallas import tpu as pltpu
from jax.experimental.pallas import tpu_sc as plsc

N = 10000
DIN = 128
DH = 32
NCLS = 2
E = 320000

NC = 2
NS = 16
NW = NC * NS
CHUNK = 128
NCH = 80
EPT = NCH * CHUNK
EPAD = NW * EPT
NBUF = 8
LOOK = 4

NDEG = 10240
DEG_SL = NDEG // NS
NACC = 10240
ROW_SL = NACC // NS

_sc_mesh = plsc.VectorSubcoreMesh(
    core_axis_name="c", subcore_axis_name="s", num_cores=NC, num_subcores=NS
)
_sc_params = pltpu.CompilerParams(use_tc_tiling_on_sc=False)


@functools.partial(
    pl.kernel,
    out_type=jax.ShapeDtypeStruct((NC, NDEG), jnp.float32),
    mesh=_sc_mesh,
    compiler_params=_sc_params,
    scratch_types=[
        pltpu.VMEM_SHARED((NDEG,), jnp.float32),
        pltpu.VMEM((NCH, CHUNK), jnp.int32),
        pltpu.VMEM((CHUNK,), jnp.float32),
        pltpu.SemaphoreType.DMA,
    ],
)
def _deg_kernel(dst_hbm, zeros1_hbm, ones_hbm, deg_out, deg_sp, dst_all, ones_v, sem):
    c = lax.axis_index("c")
    s = lax.axis_index("s")
    wid = c * NS + s
    pltpu.sync_copy(
        zeros1_hbm.at[pl.ds(s * DEG_SL, DEG_SL)],
        deg_sp.at[pl.ds(s * DEG_SL, DEG_SL)],
    )
    pltpu.sync_copy(ones_hbm, ones_v)
    pltpu.sync_copy(dst_hbm.at[wid], dst_all)
    plsc.subcore_barrier()

    def body(o, carry):
        for b in range(8):
            pltpu.async_copy(ones_v, deg_sp.at[dst_all.at[o * 8 + b]], sem, add=True)
        for b in range(8):
            pltpu.make_async_copy(ones_hbm, ones_v, sem).wait()
        return carry

    lax.fori_loop(0, NCH // 8, body, 0)
    plsc.subcore_barrier()
    pltpu.sync_copy(
        deg_sp.at[pl.ds(s * DEG_SL, DEG_SL)],
        deg_out.at[c].at[pl.ds(s * DEG_SL, DEG_SL)],
    )


@functools.partial(
    pl.kernel,
    out_type=jax.ShapeDtypeStruct((NC, NACC, DH), jnp.float32),
    mesh=_sc_mesh,
    compiler_params=_sc_params,
    scratch_types=[
        pltpu.VMEM_SHARED((NACC, DH), jnp.float32),
        pltpu.VMEM((NCH, CHUNK), jnp.int32),
        pltpu.VMEM((NCH, CHUNK), jnp.int32),
        pltpu.VMEM((NBUF, CHUNK, DH), jnp.float32),
        pltpu.SemaphoreType.DMA((NBUF,)),
    ],
)
def _msg_kernel(hs_hbm, src_hbm, dst_hbm, zeros2_hbm, acc_out,
                acc_sp, src_all, dst_all, rows, gsem):
    c = lax.axis_index("c")
    s = lax.axis_index("s")
    wid = c * NS + s
    pltpu.sync_copy(
        zeros2_hbm.at[pl.ds(s * ROW_SL, ROW_SL)],
        acc_sp.at[pl.ds(s * ROW_SL, ROW_SL)],
    )
    pltpu.sync_copy(src_hbm.at[wid], src_all)
    pltpu.sync_copy(dst_hbm.at[wid], dst_all)
    plsc.subcore_barrier()

    for k in range(LOOK):
        pltpu.async_copy(hs_hbm.at[src_all.at[k]], rows.at[k], gsem.at[k])

    def body(o, carry):
        for b in range(NBUF):
            ch = o * NBUF + b
            bf = (b + LOOK) % NBUF

            @pl.when(ch + LOOK < NCH)
            def _():
                pltpu.async_copy(
                    hs_hbm.at[src_all.at[ch + LOOK]], rows.at[bf], gsem.at[bf]
                )

            pltpu.make_async_copy(
                hs_hbm.at[pl.ds(0, CHUNK)], rows.at[b], gsem.at[b]
            ).wait()
            pltpu.sync_copy(rows.at[b], acc_sp.at[dst_all.at[ch]], add=True)
        return carry

    lax.fori_loop(0, NCH // NBUF, body, 0)
    plsc.subcore_barrier()
    pltpu.sync_copy(
        acc_sp.at[pl.ds(s * ROW_SL, ROW_SL)],
        acc_out.at[c].at[pl.ds(s * ROW_SL, ROW_SL)],
    )


def _layer1_body(x_ref, w1_ref, degcol_ref, hs1_ref, dcol_ref):
    d = lax.rsqrt(degcol_ref[...])
    h = jnp.dot(x_ref[...], w1_ref[...], preferred_element_type=jnp.float32)
    hs1_ref[...] = h * d
    dcol_ref[...] = d


def _mid_body(acc_ref, hs1_ref, dcol_ref, b1_ref, hs2_ref):
    m = (acc_ref[0, :N] + acc_ref[1, :N] + hs1_ref[...]) * dcol_ref[...]
    out1 = jnp.maximum(m + b1_ref[...], 0.0)
    hs2_ref[...] = out1 * dcol_ref[...]


def _final_body(acc_ref, hs2_ref, dcol_ref, w2_ref, b2_ref, out_ref):
    p2 = (acc_ref[0, :N] + acc_ref[1, :N] + hs2_ref[...]) * dcol_ref[...]
    logits = (
        jnp.dot(p2, w2_ref[...], preferred_element_type=jnp.float32)
        + b2_ref[...]
    )
    l0 = logits[:, 0:1]
    l1 = logits[:, 1:2]
    mx = jnp.maximum(l0, l1)
    lse = mx + jnp.log(jnp.exp(l0 - mx) + jnp.exp(l1 - mx))
    out_ref[...] = logits - lse


def kernel(x, edge_index, W1, b1, W2, b2):
    npad = EPAD - E
    pad_ar = jnp.arange(npad, dtype=jnp.int32)
    pad_src = (pad_ar * 13) % N
    pad_dst = N + pad_ar % (NACC - N)
    src = jnp.concatenate([edge_index[0], pad_src]).reshape(NW, NCH, CHUNK)
    dst = jnp.concatenate([edge_index[1], pad_dst]).reshape(NW, NCH, CHUNK)
    zeros1 = jnp.zeros((NDEG,), jnp.float32)
    zeros2 = jnp.zeros((NACC, DH), jnp.float32)
    ones = jnp.ones((CHUNK,), jnp.float32)

    degp = _deg_kernel(dst, zeros1, ones)
    degcol = (degp[0, :N] + degp[1, :N] + 1.0)[:, None]

    hs1, dcol = pl.pallas_call(
        _layer1_body,
        out_shape=[
            jax.ShapeDtypeStruct((N, DH), jnp.float32),
            jax.ShapeDtypeStruct((N, 1), jnp.float32),
        ],
    )(x, W1, degcol)

    acc1 = _msg_kernel(hs1, src, dst, zeros2)

    hs2 = pl.pallas_call(
        _mid_body,
        out_shape=jax.ShapeDtypeStruct((N, DH), jnp.float32),
    )(acc1, hs1, dcol, b1[None, :])

    acc2 = _msg_kernel(hs2, src, dst, zeros2)

    out = pl.pallas_call(
        _final_body,
        out_shape=jax.ShapeDtypeStruct((N, NCLS), jnp.float32),
    )(acc2, hs2, dcol, W2, b2[None, :])
    return out
